# baseline (speedup 1.0000x reference)
import functools

import jax
import jax.numpy as jnp
from jax import lax
from jax.experimental import pallas as pl
from jax.experimental.pallas import tpu as pltpu
from jax.experimental.pallas import tpu_sc as plsc

NC = 2
NS = 16
LANES = 16


def _edge_linear_body(ea_ref, we_ref, be_ref, out_ref):
    out_ref[...] = (
        jnp.dot(ea_ref[...], we_ref[...], preferred_element_type=jnp.float32)
        + be_ref[...]
    )


def _edge_linear(edge_attr, We, be, block=2048):
    E, DE = edge_attr.shape
    D = We.shape[1]
    assert E % block == 0
    return pl.pallas_call(
        _edge_linear_body,
        grid=(E // block,),
        in_specs=[
            pl.BlockSpec((block, DE), lambda i: (i, 0)),
            pl.BlockSpec((DE, D), lambda i: (0, 0)),
            pl.BlockSpec((1, D), lambda i: (0, 0)),
        ],
        out_specs=pl.BlockSpec((block, D), lambda i: (i, 0)),
        out_shape=jax.ShapeDtypeStruct((E, D), jnp.float32),
    )(edge_attr, We, be.reshape(1, D))


def _node_mlp_body(h_ref, a0_ref, a1_ref, w1_ref, b1_ref, w2_ref, b2_ref,
                   alpha_ref, out_ref):
    h = h_ref[...]
    s = h + a0_ref[...] + a1_ref[...]
    t = jnp.dot(s, w1_ref[...], preferred_element_type=jnp.float32) + b1_ref[...]
    t = jnp.maximum(t, 0.0)
    xn = jnp.dot(t, w2_ref[...], preferred_element_type=jnp.float32) + b2_ref[...]
    a = alpha_ref[0, 0]
    out_ref[...] = a * h + (1.0 - a) * xn


def _node_mlp(h, a0, a1, W1, b1, W2, b2, alpha, block=2000):
    Nn, D = h.shape
    assert Nn % block == 0
    return pl.pallas_call(
        _node_mlp_body,
        grid=(Nn // block,),
        in_specs=[
            pl.BlockSpec((block, D), lambda i: (i, 0)),
            pl.BlockSpec((block, D), lambda i: (i, 0)),
            pl.BlockSpec((block, D), lambda i: (i, 0)),
            pl.BlockSpec((D, D), lambda i: (0, 0)),
            pl.BlockSpec((1, D), lambda i: (0, 0)),
            pl.BlockSpec((D, D), lambda i: (0, 0)),
            pl.BlockSpec((1, D), lambda i: (0, 0)),
            pl.BlockSpec((1, 1), lambda i: (0, 0)),
        ],
        out_specs=pl.BlockSpec((block, D), lambda i: (i, 0)),
        out_shape=jax.ShapeDtypeStruct((Nn, D), jnp.float32),
    )(h, a0, a1, W1, b1.reshape(1, D), W2, b2.reshape(1, D),
      alpha.reshape(1, 1))


def _sc_message_pass(h, e, eidx, zeros_nd, chunk=16, nbuf=5):
    E, D = e.shape
    Np = zeros_nd.shape[0]
    NW = NC * NS
    per_w = E // NW
    assert per_w * NW == E and per_w % chunk == 0 and chunk % 8 == 0
    n_chunks = per_w // chunk
    assert n_chunks % nbuf == 0 and nbuf >= 3
    assert eidx.shape == (NW, n_chunks, 2, chunk)
    rows_per_sub = Np // NS
    assert rows_per_sub * NS == Np and rows_per_sub % 8 == 0

    mesh = plsc.VectorSubcoreMesh(core_axis_name="c", subcore_axis_name="s")

    @functools.partial(
        pl.kernel,
        out_type=jax.ShapeDtypeStruct((NC, Np, D), jnp.float32),
        mesh=mesh,
        scratch_types=(
            [pltpu.VMEM((chunk,), jnp.int32) for _ in range(2 * nbuf)]
            + [pltpu.VMEM((chunk, D), jnp.float32) for _ in range(2 * nbuf)]
            + [pltpu.VMEM_SHARED((Np, D), jnp.float32)]
            + [pltpu.SemaphoreType.DMA for _ in range(5 * nbuf)]
        ),
    )
    def k(h_hbm, e_hbm, eidx_hbm, z_hbm, out_hbm, *rest):
        sv = rest[0:nbuf]
        dv = rest[nbuf:2 * nbuf]
        xg = rest[2 * nbuf:3 * nbuf]
        ev = rest[3 * nbuf:4 * nbuf]
        agg_sh = rest[4 * nbuf]
        si = rest[4 * nbuf + 1:4 * nbuf + 1 + nbuf]
        sd = rest[4 * nbuf + 1 + nbuf:4 * nbuf + 1 + 2 * nbuf]
        se = rest[4 * nbuf + 1 + 2 * nbuf:4 * nbuf + 1 + 3 * nbuf]
        sg = rest[4 * nbuf + 1 + 3 * nbuf:4 * nbuf + 1 + 4 * nbuf]
        ss = rest[4 * nbuf + 1 + 4 * nbuf:4 * nbuf + 1 + 5 * nbuf]

        cid = lax.axis_index("c")
        sid = lax.axis_index("s")
        wid = cid * NS + sid
        base = wid * per_w

        def fire_ie(jj, b):
            pltpu.async_copy(eidx_hbm.at[wid, jj, 0], sv[b], si[b])
            pltpu.async_copy(eidx_hbm.at[wid, jj, 1], dv[b], sd[b])
            pltpu.async_copy(e_hbm.at[pl.ds(base + jj * chunk, chunk), :],
                             ev[b], se[b])

        def fire_gather(jj, b):
            pltpu.make_async_copy(eidx_hbm.at[wid, jj, 0], sv[b], si[b]).wait()
            pltpu.async_copy(h_hbm.at[sv[b]], xg[b], sg[b])

        for c in range(nbuf - 1):
            fire_ie(c, c)
        for c in range(2):
            fire_gather(c, c)

        r0 = sid * rows_per_sub
        pltpu.sync_copy(z_hbm.at[pl.ds(r0, rows_per_sub), :],
                        agg_sh.at[pl.ds(r0, rows_per_sub), :])
        plsc.subcore_barrier()

        @pl.loop(0, n_chunks, step=nbuf)
        def _(j0):
            for b in range(nbuf):
                jj = j0 + b
                bm1 = (b + nbuf - 1) % nbuf
                b2 = (b + 2) % nbuf

                @pl.when(jj >= 1)
                def _():
                    pltpu.make_async_copy(
                        xg[bm1], agg_sh.at[dv[bm1]], ss[bm1]).wait()

                @pl.when(jj + nbuf - 1 < n_chunks)
                def _():
                    fire_ie(jj + nbuf - 1, bm1)

                @pl.when(jj + 2 < n_chunks)
                def _():
                    fire_gather(jj + 2, b2)

                pltpu.make_async_copy(
                    e_hbm.at[pl.ds(base, chunk), :], ev[b], se[b]).wait()
                pltpu.make_async_copy(
                    eidx_hbm.at[wid, jj, 1], dv[b], sd[b]).wait()
                pltpu.make_async_copy(
                    h_hbm.at[sv[b]], xg[b], sg[b]).wait()

                @plsc.parallel_loop(0, chunk)
                def _(r):
                    for c0 in range(0, D, LANES):
                        v = xg[b][r, pl.ds(c0, LANES)] \
                            + ev[b][r, pl.ds(c0, LANES)]
                        xg[b][r, pl.ds(c0, LANES)] = jnp.maximum(v, 0.0)

                pltpu.async_copy(xg[b], agg_sh.at[dv[b]], ss[b],
                                 add=True)

        lb = (n_chunks - 1) % nbuf
        pltpu.make_async_copy(
            xg[lb], agg_sh.at[dv[lb]], ss[lb]).wait()

        plsc.subcore_barrier()
        pltpu.sync_copy(agg_sh.at[pl.ds(r0, rows_per_sub), :],
                        out_hbm.at[cid, pl.ds(r0, rows_per_sub), :])

    return k(h, e, eidx, zeros_nd)


def kernel(x, edge_index, edge_attr, params):
    chunk, nbuf = 32, 5
    Nn, D = x.shape
    E = edge_index.shape[1]
    NW = NC * NS
    Np = ((Nn + 8 * NS - 1) // (8 * NS)) * (8 * NS)
    step = chunk * nbuf * 2
    per_w = -(-E // (NW * step)) * step
    Ep = NW * per_w
    if Ep > E:
        edge_index = jnp.concatenate(
            [edge_index,
             jnp.concatenate(
                 [jnp.zeros((1, Ep - E), jnp.int32),
                  jnp.full((1, Ep - E), Nn, jnp.int32)])], axis=1)
        edge_attr = jnp.concatenate(
            [edge_attr, jnp.zeros((Ep - E, edge_attr.shape[1]),
                                  edge_attr.dtype)])
    n_chunks = per_w // chunk
    eidx = edge_index.reshape(2, NW, n_chunks, chunk).transpose(1, 2, 0, 3)
    zeros_nd = jnp.zeros((Np, D), jnp.float32)

    es = [_edge_linear(edge_attr, We, be) for (We, be, *_rest) in params]

    h = x
    xs = [x]
    for l, (We, be, W1, b1, W2, b2, alpha) in enumerate(params):
        agg2 = _sc_message_pass(h, es[l], eidx, zeros_nd,
                                chunk=chunk, nbuf=nbuf)
        h = _node_mlp(h, agg2[0], agg2[1], W1, b1, W2, b2, alpha)
        xs.append(h)
    return jnp.concatenate(xs, axis=-1)


# speedup vs baseline: 1.0480x; 1.0480x over previous
import functools

import jax
import jax.numpy as jnp
from jax import lax
from jax.experimental import pallas as pl
from jax.experimental.pallas import tpu as pltpu
from jax.experimental.pallas import tpu_sc as plsc

NC = 2
NS = 16
LANES = 16


def _edge_linear_body(ea_ref, we_ref, be_ref, out_ref):
    out_ref[...] = (
        jnp.dot(ea_ref[...], we_ref[...], preferred_element_type=jnp.float32)
        + be_ref[...]
    )


def _edge_linear(edge_attr, We, be, block=2048):
    E, DE = edge_attr.shape
    D = We.shape[1]
    assert E % block == 0
    return pl.pallas_call(
        _edge_linear_body,
        grid=(E // block,),
        in_specs=[
            pl.BlockSpec((block, DE), lambda i: (i, 0)),
            pl.BlockSpec((DE, D), lambda i: (0, 0)),
            pl.BlockSpec((1, D), lambda i: (0, 0)),
        ],
        out_specs=pl.BlockSpec((block, D), lambda i: (i, 0)),
        out_shape=jax.ShapeDtypeStruct((E, D), jnp.float32),
    )(edge_attr, We, be.reshape(1, D))


def _node_mlp_body(h_ref, a0_ref, a1_ref, w1_ref, b1_ref, w2_ref, b2_ref,
                   alpha_ref, out_ref):
    h = h_ref[...]
    s = h + a0_ref[...] + a1_ref[...]
    t = jnp.dot(s, w1_ref[...], preferred_element_type=jnp.float32) + b1_ref[...]
    t = jnp.maximum(t, 0.0)
    xn = jnp.dot(t, w2_ref[...], preferred_element_type=jnp.float32) + b2_ref[...]
    a = alpha_ref[0, 0]
    out_ref[...] = a * h + (1.0 - a) * xn


def _node_mlp(h, a0, a1, W1, b1, W2, b2, alpha, block=2000):
    Nn, D = h.shape
    assert Nn % block == 0
    return pl.pallas_call(
        _node_mlp_body,
        grid=(Nn // block,),
        in_specs=[
            pl.BlockSpec((block, D), lambda i: (i, 0)),
            pl.BlockSpec((block, D), lambda i: (i, 0)),
            pl.BlockSpec((block, D), lambda i: (i, 0)),
            pl.BlockSpec((D, D), lambda i: (0, 0)),
            pl.BlockSpec((1, D), lambda i: (0, 0)),
            pl.BlockSpec((D, D), lambda i: (0, 0)),
            pl.BlockSpec((1, D), lambda i: (0, 0)),
            pl.BlockSpec((1, 1), lambda i: (0, 0)),
        ],
        out_specs=pl.BlockSpec((block, D), lambda i: (i, 0)),
        out_shape=jax.ShapeDtypeStruct((Nn, D), jnp.float32),
    )(h, a0, a1, W1, b1.reshape(1, D), W2, b2.reshape(1, D),
      alpha.reshape(1, 1))


def _sc_message_pass(h, e, eidx, zeros_nd, chunk=16, nbuf=5):
    E, D = e.shape
    Np = zeros_nd.shape[0]
    NW = NC * NS
    per_w = E // NW
    assert per_w * NW == E and per_w % chunk == 0 and chunk % 8 == 0
    n_chunks = per_w // chunk
    assert n_chunks % nbuf == 0 and nbuf >= 3
    assert eidx.shape == (NW, n_chunks, 2, chunk)
    rows_per_sub = Np // NS
    assert rows_per_sub * NS == Np and rows_per_sub % 8 == 0

    mesh = plsc.VectorSubcoreMesh(core_axis_name="c", subcore_axis_name="s")

    @functools.partial(
        pl.kernel,
        out_type=jax.ShapeDtypeStruct((NC, Np, D), jnp.float32),
        mesh=mesh,
        scratch_types=(
            [pltpu.VMEM((chunk,), jnp.int32) for _ in range(2 * nbuf)]
            + [pltpu.VMEM((chunk, D), jnp.float32) for _ in range(2 * nbuf)]
            + [pltpu.VMEM_SHARED((Np, D), jnp.float32)]
            + [pltpu.SemaphoreType.DMA for _ in range(5 * nbuf)]
        ),
    )
    def k(h_hbm, e_hbm, eidx_hbm, z_hbm, out_hbm, *rest):
        sv = rest[0:nbuf]
        dv = rest[nbuf:2 * nbuf]
        xg = rest[2 * nbuf:3 * nbuf]
        ev = rest[3 * nbuf:4 * nbuf]
        agg_sh = rest[4 * nbuf]
        si = rest[4 * nbuf + 1:4 * nbuf + 1 + nbuf]
        sd = rest[4 * nbuf + 1 + nbuf:4 * nbuf + 1 + 2 * nbuf]
        se = rest[4 * nbuf + 1 + 2 * nbuf:4 * nbuf + 1 + 3 * nbuf]
        sg = rest[4 * nbuf + 1 + 3 * nbuf:4 * nbuf + 1 + 4 * nbuf]
        ss = rest[4 * nbuf + 1 + 4 * nbuf:4 * nbuf + 1 + 5 * nbuf]

        cid = lax.axis_index("c")
        sid = lax.axis_index("s")
        wid = cid * NS + sid
        base = wid * per_w

        def fire_ie(jj, b):
            pltpu.async_copy(eidx_hbm.at[wid, jj, 0], sv[b], si[b])
            pltpu.async_copy(eidx_hbm.at[wid, jj, 1], dv[b], sd[b])
            pltpu.async_copy(e_hbm.at[pl.ds(base + jj * chunk, chunk), :],
                             ev[b], se[b])

        def fire_gather(jj, b):
            pltpu.make_async_copy(eidx_hbm.at[wid, jj, 0], sv[b], si[b]).wait()
            pltpu.async_copy(h_hbm.at[sv[b]], xg[b], sg[b])

        for c in range(nbuf - 1):
            fire_ie(c, c)
        for c in range(2):
            fire_gather(c, c)

        r0 = sid * rows_per_sub
        pltpu.sync_copy(z_hbm.at[pl.ds(r0, rows_per_sub), :],
                        agg_sh.at[pl.ds(r0, rows_per_sub), :])
        plsc.subcore_barrier()

        @pl.loop(0, n_chunks, step=nbuf)
        def _(j0):
            for b in range(nbuf):
                jj = j0 + b
                bm1 = (b + nbuf - 1) % nbuf
                b2 = (b + 2) % nbuf

                @pl.when(jj >= 1)
                def _():
                    pltpu.make_async_copy(
                        xg[bm1], agg_sh.at[dv[bm1]], ss[bm1]).wait()

                @pl.when(jj + nbuf - 1 < n_chunks)
                def _():
                    fire_ie(jj + nbuf - 1, bm1)

                @pl.when(jj + 2 < n_chunks)
                def _():
                    fire_gather(jj + 2, b2)

                pltpu.make_async_copy(
                    e_hbm.at[pl.ds(base, chunk), :], ev[b], se[b]).wait()
                pltpu.make_async_copy(
                    eidx_hbm.at[wid, jj, 1], dv[b], sd[b]).wait()
                pltpu.make_async_copy(
                    h_hbm.at[sv[b]], xg[b], sg[b]).wait()

                @plsc.parallel_loop(0, chunk)
                def _(r):
                    for c0 in range(0, D, LANES):
                        v = xg[b][r, pl.ds(c0, LANES)] \
                            + ev[b][r, pl.ds(c0, LANES)]
                        xg[b][r, pl.ds(c0, LANES)] = jnp.maximum(v, 0.0)

                pltpu.async_copy(xg[b], agg_sh.at[dv[b]], ss[b],
                                 add=True)

        lb = (n_chunks - 1) % nbuf
        pltpu.make_async_copy(
            xg[lb], agg_sh.at[dv[lb]], ss[lb]).wait()

        plsc.subcore_barrier()
        pltpu.sync_copy(agg_sh.at[pl.ds(r0, rows_per_sub), :],
                        out_hbm.at[cid, pl.ds(r0, rows_per_sub), :])

    return k(h, e, eidx, zeros_nd)


def kernel(x, edge_index, edge_attr, params):
    chunk, nbuf = 16, 5
    Nn, D = x.shape
    E = edge_index.shape[1]
    NW = NC * NS
    Np = ((Nn + 8 * NS - 1) // (8 * NS)) * (8 * NS)
    step = chunk * nbuf
    while (NW * step) % 2048:
        step *= 2
    per_w = -(-E // (NW * step)) * step
    Ep = NW * per_w
    if Ep > E:
        edge_index = jnp.concatenate(
            [edge_index,
             jnp.concatenate(
                 [jnp.zeros((1, Ep - E), jnp.int32),
                  jnp.full((1, Ep - E), Nn, jnp.int32)])], axis=1)
        edge_attr = jnp.concatenate(
            [edge_attr, jnp.zeros((Ep - E, edge_attr.shape[1]),
                                  edge_attr.dtype)])
    n_chunks = per_w // chunk
    eidx = edge_index.reshape(2, NW, n_chunks, chunk).transpose(1, 2, 0, 3)
    zeros_nd = jnp.zeros((Np, D), jnp.float32)

    es = [_edge_linear(edge_attr, We, be) for (We, be, *_rest) in params]

    h = x
    xs = [x]
    for l, (We, be, W1, b1, W2, b2, alpha) in enumerate(params):
        agg2 = _sc_message_pass(h, es[l], eidx, zeros_nd,
                                chunk=chunk, nbuf=nbuf)
        h = _node_mlp(h, agg2[0], agg2[1], W1, b1, W2, b2, alpha)
        xs.append(h)
    return jnp.concatenate(xs, axis=-1)


# speedup vs baseline: 1.7844x; 1.7027x over previous
import functools

import jax
import jax.numpy as jnp
from jax import lax
from jax.experimental import pallas as pl
from jax.experimental.pallas import tpu as pltpu
from jax.experimental.pallas import tpu_sc as plsc

NC = 2
NS = 16
LANES = 16


def _edge_linear_body(ea_ref, we_ref, be_ref, out_ref):
    out_ref[...] = (
        jnp.dot(ea_ref[...], we_ref[...], preferred_element_type=jnp.float32)
        + be_ref[...]
    )


def _edge_linear(edge_attr, We, be, block=2048):
    E, DE = edge_attr.shape
    D = We.shape[1]
    assert E % block == 0
    return pl.pallas_call(
        _edge_linear_body,
        grid=(E // block,),
        in_specs=[
            pl.BlockSpec((block, DE), lambda i: (i, 0)),
            pl.BlockSpec((DE, D), lambda i: (0, 0)),
            pl.BlockSpec((1, D), lambda i: (0, 0)),
        ],
        out_specs=pl.BlockSpec((block, D), lambda i: (i, 0)),
        out_shape=jax.ShapeDtypeStruct((E, D), jnp.float32),
    )(edge_attr, We, be.reshape(1, D))


def _node_mlp_body(h_ref, a0_ref, a1_ref, w1_ref, b1_ref, w2_ref, b2_ref,
                   alpha_ref, out_ref):
    h = h_ref[...]
    s = h + a0_ref[...] + a1_ref[...]
    t = jnp.dot(s, w1_ref[...], preferred_element_type=jnp.float32) + b1_ref[...]
    t = jnp.maximum(t, 0.0)
    xn = jnp.dot(t, w2_ref[...], preferred_element_type=jnp.float32) + b2_ref[...]
    a = alpha_ref[0, 0]
    out_ref[...] = a * h + (1.0 - a) * xn


def _node_mlp(h, a0, a1, W1, b1, W2, b2, alpha, block=2000):
    Nn, D = h.shape
    assert Nn % block == 0
    return pl.pallas_call(
        _node_mlp_body,
        grid=(Nn // block,),
        in_specs=[
            pl.BlockSpec((block, D), lambda i: (i, 0)),
            pl.BlockSpec((block, D), lambda i: (i, 0)),
            pl.BlockSpec((block, D), lambda i: (i, 0)),
            pl.BlockSpec((D, D), lambda i: (0, 0)),
            pl.BlockSpec((1, D), lambda i: (0, 0)),
            pl.BlockSpec((D, D), lambda i: (0, 0)),
            pl.BlockSpec((1, D), lambda i: (0, 0)),
            pl.BlockSpec((1, 1), lambda i: (0, 0)),
        ],
        out_specs=pl.BlockSpec((block, D), lambda i: (i, 0)),
        out_shape=jax.ShapeDtypeStruct((Nn, D), jnp.float32),
    )(h, a0, a1, W1, b1.reshape(1, D), W2, b2.reshape(1, D),
      alpha.reshape(1, 1))


def _sc_message_pass(h, e, eidx, zeros_nd, chunk=16, nbuf=5):
    E, D = e.shape
    Np = zeros_nd.shape[0]
    NW = NC * NS
    per_w = E // NW
    assert per_w * NW == E and per_w % chunk == 0 and chunk % 8 == 0
    n_chunks = per_w // chunk
    assert n_chunks % nbuf == 0 and nbuf >= 3
    assert eidx.shape == (NW, n_chunks, 2, chunk)
    rows_per_sub = Np // NS
    assert rows_per_sub * NS == Np and rows_per_sub % 8 == 0

    mesh = plsc.VectorSubcoreMesh(core_axis_name="c", subcore_axis_name="s")

    @functools.partial(
        pl.kernel,
        out_type=jax.ShapeDtypeStruct((NC, Np, D), jnp.float32),
        mesh=mesh,
        scratch_types=(
            [pltpu.VMEM((chunk,), jnp.int32) for _ in range(2 * nbuf)]
            + [pltpu.VMEM((chunk, D), jnp.float32) for _ in range(2 * nbuf)]
            + [pltpu.VMEM_SHARED((Np, D), jnp.float32)]
            + [pltpu.SemaphoreType.DMA for _ in range(5 * nbuf)]
        ),
    )
    def k(h_hbm, e_hbm, eidx_hbm, z_hbm, out_hbm, *rest):
        sv = rest[0:nbuf]
        dv = rest[nbuf:2 * nbuf]
        xg = rest[2 * nbuf:3 * nbuf]
        ev = rest[3 * nbuf:4 * nbuf]
        agg_sh = rest[4 * nbuf]
        si = rest[4 * nbuf + 1:4 * nbuf + 1 + nbuf]
        sd = rest[4 * nbuf + 1 + nbuf:4 * nbuf + 1 + 2 * nbuf]
        se = rest[4 * nbuf + 1 + 2 * nbuf:4 * nbuf + 1 + 3 * nbuf]
        sg = rest[4 * nbuf + 1 + 3 * nbuf:4 * nbuf + 1 + 4 * nbuf]
        ss = rest[4 * nbuf + 1 + 4 * nbuf:4 * nbuf + 1 + 5 * nbuf]

        cid = lax.axis_index("c")
        sid = lax.axis_index("s")
        wid = cid * NS + sid
        base = wid * per_w

        def fire_ie(jj, b):
            pltpu.async_copy(eidx_hbm.at[wid, jj, 0], sv[b], si[b])
            pltpu.async_copy(eidx_hbm.at[wid, jj, 1], dv[b], sd[b])
            pltpu.async_copy(e_hbm.at[pl.ds(base + jj * chunk, chunk), :],
                             ev[b], se[b])

        def fire_gather(jj, b):
            pltpu.make_async_copy(eidx_hbm.at[wid, jj, 0], sv[b], si[b]).wait()
            pltpu.async_copy(h_hbm.at[sv[b]], xg[b], sg[b])

        for c in range(nbuf - 1):
            fire_ie(c, c)
        for c in range(2):
            fire_gather(c, c)

        r0 = sid * rows_per_sub
        pltpu.sync_copy(z_hbm.at[pl.ds(r0, rows_per_sub), :],
                        agg_sh.at[pl.ds(r0, rows_per_sub), :])
        plsc.subcore_barrier()

        @pl.loop(0, n_chunks, step=nbuf)
        def _(j0):
            for b in range(nbuf):
                jj = j0 + b
                bm1 = (b + nbuf - 1) % nbuf
                b2 = (b + 2) % nbuf

                @pl.when(jj >= 1)
                def _():
                    pltpu.make_async_copy(
                        xg[bm1], agg_sh.at[dv[bm1]], ss[bm1]).wait()

                @pl.when(jj + nbuf - 1 < n_chunks)
                def _():
                    fire_ie(jj + nbuf - 1, bm1)

                @pl.when(jj + 2 < n_chunks)
                def _():
                    fire_gather(jj + 2, b2)

                pltpu.make_async_copy(
                    e_hbm.at[pl.ds(base, chunk), :], ev[b], se[b]).wait()
                pltpu.make_async_copy(
                    eidx_hbm.at[wid, jj, 1], dv[b], sd[b]).wait()
                pltpu.make_async_copy(
                    h_hbm.at[sv[b]], xg[b], sg[b]).wait()

                @plsc.parallel_loop(0, chunk)
                def _(r):
                    for c0 in range(0, D, LANES):
                        v = xg[b][r, pl.ds(c0, LANES)] \
                            + ev[b][r, pl.ds(c0, LANES)]
                        xg[b][r, pl.ds(c0, LANES)] = jnp.maximum(v, 0.0)

                pltpu.async_copy(xg[b], agg_sh.at[dv[b]], ss[b],
                                 add=True)

        lb = (n_chunks - 1) % nbuf
        pltpu.make_async_copy(
            xg[lb], agg_sh.at[dv[lb]], ss[lb]).wait()

        plsc.subcore_barrier()
        pltpu.sync_copy(agg_sh.at[pl.ds(r0, rows_per_sub), :],
                        out_hbm.at[cid, pl.ds(r0, rows_per_sub), :])

    return k(h, e, eidx, zeros_nd)


def kernel(x, edge_index, edge_attr, params):
    chunk, nbuf = 16, 5
    Nn, D = x.shape
    E = edge_index.shape[1]
    NW = NC * NS
    Np = ((Nn + 8 * NS - 1) // (8 * NS)) * (8 * NS)
    step = chunk * nbuf
    while (NW * step) % 2048:
        step *= 2
    per_w = -(-E // (NW * step)) * step
    Ep = NW * per_w
    if Ep > E:
        pad_ar = jnp.arange(Ep - E, dtype=jnp.int32)
        edge_index = jnp.concatenate(
            [edge_index,
             jnp.stack([pad_ar % Nn, Nn + pad_ar % (Np - Nn)])], axis=1)
        edge_attr = jnp.concatenate(
            [edge_attr, jnp.zeros((Ep - E, edge_attr.shape[1]),
                                  edge_attr.dtype)])
    n_chunks = per_w // chunk
    eidx = edge_index.reshape(2, NW, n_chunks, chunk).transpose(1, 2, 0, 3)
    zeros_nd = jnp.zeros((Np, D), jnp.float32)

    es = [_edge_linear(edge_attr, We, be) for (We, be, *_rest) in params]

    h = x
    xs = [x]
    for l, (We, be, W1, b1, W2, b2, alpha) in enumerate(params):
        agg2 = _sc_message_pass(h, es[l], eidx, zeros_nd,
                                chunk=chunk, nbuf=nbuf)
        h = _node_mlp(h, agg2[0], agg2[1], W1, b1, W2, b2, alpha)
        xs.append(h)
    return jnp.concatenate(xs, axis=-1)


# speedup vs baseline: 2.1320x; 1.1948x over previous
import functools

import jax
import jax.numpy as jnp
from jax import lax
from jax.experimental import pallas as pl
from jax.experimental.pallas import tpu as pltpu
from jax.experimental.pallas import tpu_sc as plsc

NC = 2
NS = 16
LANES = 16


def _edge_linear_body(ea_ref, we_ref, be_ref, out_ref):
    out_ref[...] = (
        jnp.dot(ea_ref[...], we_ref[...], preferred_element_type=jnp.float32)
        + be_ref[...]
    )


def _edge_linear(edge_attr, We, be, block=2048):
    E, DE = edge_attr.shape
    D = We.shape[1]
    assert E % block == 0
    return pl.pallas_call(
        _edge_linear_body,
        grid=(E // block,),
        in_specs=[
            pl.BlockSpec((block, DE), lambda i: (i, 0)),
            pl.BlockSpec((DE, D), lambda i: (0, 0)),
            pl.BlockSpec((1, D), lambda i: (0, 0)),
        ],
        out_specs=pl.BlockSpec((block, D), lambda i: (i, 0)),
        out_shape=jax.ShapeDtypeStruct((E, D), jnp.float32),
    )(edge_attr, We, be.reshape(1, D))


def _node_mlp_body(h_ref, a0_ref, a1_ref, w1_ref, b1_ref, w2_ref, b2_ref,
                   alpha_ref, out_ref):
    h = h_ref[...]
    s = h + a0_ref[...] + a1_ref[...]
    t = jnp.dot(s, w1_ref[...], preferred_element_type=jnp.float32) + b1_ref[...]
    t = jnp.maximum(t, 0.0)
    xn = jnp.dot(t, w2_ref[...], preferred_element_type=jnp.float32) + b2_ref[...]
    a = alpha_ref[0, 0]
    out_ref[...] = a * h + (1.0 - a) * xn


def _node_mlp(h, a0, a1, W1, b1, W2, b2, alpha, block=2000):
    Nn, D = h.shape
    assert Nn % block == 0
    return pl.pallas_call(
        _node_mlp_body,
        grid=(Nn // block,),
        in_specs=[
            pl.BlockSpec((block, D), lambda i: (i, 0)),
            pl.BlockSpec((block, D), lambda i: (i, 0)),
            pl.BlockSpec((block, D), lambda i: (i, 0)),
            pl.BlockSpec((D, D), lambda i: (0, 0)),
            pl.BlockSpec((1, D), lambda i: (0, 0)),
            pl.BlockSpec((D, D), lambda i: (0, 0)),
            pl.BlockSpec((1, D), lambda i: (0, 0)),
            pl.BlockSpec((1, 1), lambda i: (0, 0)),
        ],
        out_specs=pl.BlockSpec((block, D), lambda i: (i, 0)),
        out_shape=jax.ShapeDtypeStruct((Nn, D), jnp.float32),
    )(h, a0, a1, W1, b1.reshape(1, D), W2, b2.reshape(1, D),
      alpha.reshape(1, 1))


def _sc_message_pass(h, e, eidx, zeros_nd, chunk=16, nbuf=5):
    E, D = e.shape
    Np = zeros_nd.shape[0]
    NW = NC * NS
    per_w = E // NW
    assert per_w * NW == E and per_w % chunk == 0 and chunk % 8 == 0
    n_chunks = per_w // chunk
    assert n_chunks % nbuf == 0 and nbuf >= 3
    assert eidx.shape == (NW, n_chunks, 2, chunk)
    rows_per_sub = Np // NS
    assert rows_per_sub * NS == Np and rows_per_sub % 8 == 0

    mesh = plsc.VectorSubcoreMesh(core_axis_name="c", subcore_axis_name="s")

    @functools.partial(
        pl.kernel,
        out_type=jax.ShapeDtypeStruct((NC, Np, D), jnp.float32),
        mesh=mesh,
        scratch_types=(
            [pltpu.VMEM((chunk,), jnp.int32) for _ in range(2 * nbuf)]
            + [pltpu.VMEM((chunk, D), jnp.float32) for _ in range(2 * nbuf)]
            + [pltpu.VMEM_SHARED((Np, D), jnp.float32)]
            + [pltpu.SemaphoreType.DMA for _ in range(5 * nbuf)]
        ),
    )
    def k(h_hbm, e_hbm, eidx_hbm, z_hbm, out_hbm, *rest):
        sv = rest[0:nbuf]
        dv = rest[nbuf:2 * nbuf]
        xg = rest[2 * nbuf:3 * nbuf]
        ev = rest[3 * nbuf:4 * nbuf]
        agg_sh = rest[4 * nbuf]
        si = rest[4 * nbuf + 1:4 * nbuf + 1 + nbuf]
        sd = rest[4 * nbuf + 1 + nbuf:4 * nbuf + 1 + 2 * nbuf]
        se = rest[4 * nbuf + 1 + 2 * nbuf:4 * nbuf + 1 + 3 * nbuf]
        sg = rest[4 * nbuf + 1 + 3 * nbuf:4 * nbuf + 1 + 4 * nbuf]
        ss = rest[4 * nbuf + 1 + 4 * nbuf:4 * nbuf + 1 + 5 * nbuf]

        cid = lax.axis_index("c")
        sid = lax.axis_index("s")
        wid = cid * NS + sid
        base = wid * per_w

        def fire_ie(jj, b):
            pltpu.async_copy(eidx_hbm.at[wid, jj, 0], sv[b], si[b])
            pltpu.async_copy(eidx_hbm.at[wid, jj, 1], dv[b], sd[b])
            pltpu.async_copy(e_hbm.at[pl.ds(base + jj * chunk, chunk), :],
                             ev[b], se[b])

        def fire_gather(jj, b):
            pltpu.make_async_copy(eidx_hbm.at[wid, jj, 0], sv[b], si[b]).wait()
            pltpu.async_copy(h_hbm.at[sv[b]], xg[b], sg[b])

        for c in range(nbuf - 1):
            fire_ie(c, c)
        for c in range(2):
            fire_gather(c, c)

        r0 = sid * rows_per_sub
        pltpu.sync_copy(z_hbm.at[pl.ds(r0, rows_per_sub), :],
                        agg_sh.at[pl.ds(r0, rows_per_sub), :])
        plsc.subcore_barrier()

        @pl.loop(0, n_chunks, step=nbuf)
        def _(j0):
            for b in range(nbuf):
                jj = j0 + b
                bm1 = (b + nbuf - 1) % nbuf
                b2 = (b + 2) % nbuf

                @pl.when(jj >= 1)
                def _():
                    pltpu.make_async_copy(
                        xg[bm1], agg_sh.at[dv[bm1]], ss[bm1]).wait()

                @pl.when(jj + nbuf - 1 < n_chunks)
                def _():
                    fire_ie(jj + nbuf - 1, bm1)

                @pl.when(jj + 2 < n_chunks)
                def _():
                    fire_gather(jj + 2, b2)

                pltpu.make_async_copy(
                    e_hbm.at[pl.ds(base, chunk), :], ev[b], se[b]).wait()
                pltpu.make_async_copy(
                    eidx_hbm.at[wid, jj, 1], dv[b], sd[b]).wait()
                pltpu.make_async_copy(
                    h_hbm.at[sv[b]], xg[b], sg[b]).wait()

                @plsc.parallel_loop(0, chunk)
                def _(r):
                    for c0 in range(0, D, LANES):
                        v = xg[b][r, pl.ds(c0, LANES)] \
                            + ev[b][r, pl.ds(c0, LANES)]
                        xg[b][r, pl.ds(c0, LANES)] = jnp.maximum(v, 0.0)

                pltpu.async_copy(xg[b], agg_sh.at[dv[b]], ss[b],
                                 add=True)

        lb = (n_chunks - 1) % nbuf
        pltpu.make_async_copy(
            xg[lb], agg_sh.at[dv[lb]], ss[lb]).wait()

        plsc.subcore_barrier()
        pltpu.sync_copy(agg_sh.at[pl.ds(r0, rows_per_sub), :],
                        out_hbm.at[cid, pl.ds(r0, rows_per_sub), :])

    return k(h, e, eidx, zeros_nd)


def kernel(x, edge_index, edge_attr, params):
    chunk, nbuf = 32, 5
    Nn, D = x.shape
    E = edge_index.shape[1]
    NW = NC * NS
    Np = ((Nn + 8 * NS - 1) // (8 * NS)) * (8 * NS)
    step = chunk * nbuf
    while (NW * step) % 2048:
        step *= 2
    per_w = -(-E // (NW * step)) * step
    Ep = NW * per_w
    if Ep > E:
        pad_ar = jnp.arange(Ep - E, dtype=jnp.int32)
        edge_index = jnp.concatenate(
            [edge_index,
             jnp.stack([pad_ar % Nn, Nn + pad_ar % (Np - Nn)])], axis=1)
        edge_attr = jnp.concatenate(
            [edge_attr, jnp.zeros((Ep - E, edge_attr.shape[1]),
                                  edge_attr.dtype)])
    n_chunks = per_w // chunk
    eidx = edge_index.reshape(2, NW, n_chunks, chunk).transpose(1, 2, 0, 3)
    zeros_nd = jnp.zeros((Np, D), jnp.float32)

    es = [_edge_linear(edge_attr, We, be) for (We, be, *_rest) in params]

    h = x
    xs = [x]
    for l, (We, be, W1, b1, W2, b2, alpha) in enumerate(params):
        agg2 = _sc_message_pass(h, es[l], eidx, zeros_nd,
                                chunk=chunk, nbuf=nbuf)
        h = _node_mlp(h, agg2[0], agg2[1], W1, b1, W2, b2, alpha)
        xs.append(h)
    return jnp.concatenate(xs, axis=-1)


# speedup vs baseline: 2.1572x; 1.0118x over previous
import functools

import jax
import jax.numpy as jnp
from jax import lax
from jax.experimental import pallas as pl
from jax.experimental.pallas import tpu as pltpu
from jax.experimental.pallas import tpu_sc as plsc

NC = 2
NS = 16
LANES = 16


def _edge_linear_body(ea_ref, we_ref, be_ref, out_ref):
    out_ref[...] = (
        jnp.dot(ea_ref[...], we_ref[...], preferred_element_type=jnp.float32)
        + be_ref[...]
    )


def _edge_linear(edge_attr, We, be, block=2048):
    E, DE = edge_attr.shape
    D = We.shape[1]
    assert E % block == 0
    return pl.pallas_call(
        _edge_linear_body,
        grid=(E // block,),
        in_specs=[
            pl.BlockSpec((block, DE), lambda i: (i, 0)),
            pl.BlockSpec((DE, D), lambda i: (0, 0)),
            pl.BlockSpec((1, D), lambda i: (0, 0)),
        ],
        out_specs=pl.BlockSpec((block, D), lambda i: (i, 0)),
        out_shape=jax.ShapeDtypeStruct((E, D), jnp.float32),
    )(edge_attr, We, be.reshape(1, D))


def _node_mlp_body(h_ref, a0_ref, a1_ref, w1_ref, b1_ref, w2_ref, b2_ref,
                   alpha_ref, out_ref):
    h = h_ref[...]
    s = h + a0_ref[...] + a1_ref[...]
    t = jnp.dot(s, w1_ref[...], preferred_element_type=jnp.float32) + b1_ref[...]
    t = jnp.maximum(t, 0.0)
    xn = jnp.dot(t, w2_ref[...], preferred_element_type=jnp.float32) + b2_ref[...]
    a = alpha_ref[0, 0]
    out_ref[...] = a * h + (1.0 - a) * xn


def _node_mlp(h, a0, a1, W1, b1, W2, b2, alpha, block=2000):
    Nn, D = h.shape
    assert Nn % block == 0
    return pl.pallas_call(
        _node_mlp_body,
        grid=(Nn // block,),
        in_specs=[
            pl.BlockSpec((block, D), lambda i: (i, 0)),
            pl.BlockSpec((block, D), lambda i: (i, 0)),
            pl.BlockSpec((block, D), lambda i: (i, 0)),
            pl.BlockSpec((D, D), lambda i: (0, 0)),
            pl.BlockSpec((1, D), lambda i: (0, 0)),
            pl.BlockSpec((D, D), lambda i: (0, 0)),
            pl.BlockSpec((1, D), lambda i: (0, 0)),
            pl.BlockSpec((1, 1), lambda i: (0, 0)),
        ],
        out_specs=pl.BlockSpec((block, D), lambda i: (i, 0)),
        out_shape=jax.ShapeDtypeStruct((Nn, D), jnp.float32),
    )(h, a0, a1, W1, b1.reshape(1, D), W2, b2.reshape(1, D),
      alpha.reshape(1, 1))


def _sc_message_pass(h, e, eidx, zeros_nd, chunk=16, nbuf=5):
    E, D = e.shape
    Np = zeros_nd.shape[0]
    NW = NC * NS
    per_w = E // NW
    assert per_w * NW == E and per_w % chunk == 0 and chunk % 8 == 0
    n_chunks = per_w // chunk
    assert n_chunks % nbuf == 0 and nbuf >= 3
    assert eidx.shape == (NW, n_chunks, 2, chunk)
    rows_per_sub = Np // NS
    assert rows_per_sub * NS == Np and rows_per_sub % 8 == 0

    mesh = plsc.VectorSubcoreMesh(core_axis_name="c", subcore_axis_name="s")

    @functools.partial(
        pl.kernel,
        out_type=jax.ShapeDtypeStruct((NC, Np, D), jnp.float32),
        mesh=mesh,
        scratch_types=(
            [pltpu.VMEM((chunk,), jnp.int32) for _ in range(2 * nbuf)]
            + [pltpu.VMEM((chunk, D), jnp.float32) for _ in range(2 * nbuf)]
            + [pltpu.VMEM_SHARED((Np, D), jnp.float32)]
            + [pltpu.SemaphoreType.DMA for _ in range(5 * nbuf)]
        ),
    )
    def k(h_hbm, e_hbm, eidx_hbm, z_hbm, out_hbm, *rest):
        sv = rest[0:nbuf]
        dv = rest[nbuf:2 * nbuf]
        xg = rest[2 * nbuf:3 * nbuf]
        ev = rest[3 * nbuf:4 * nbuf]
        agg_sh = rest[4 * nbuf]
        si = rest[4 * nbuf + 1:4 * nbuf + 1 + nbuf]
        sd = rest[4 * nbuf + 1 + nbuf:4 * nbuf + 1 + 2 * nbuf]
        se = rest[4 * nbuf + 1 + 2 * nbuf:4 * nbuf + 1 + 3 * nbuf]
        sg = rest[4 * nbuf + 1 + 3 * nbuf:4 * nbuf + 1 + 4 * nbuf]
        ss = rest[4 * nbuf + 1 + 4 * nbuf:4 * nbuf + 1 + 5 * nbuf]

        cid = lax.axis_index("c")
        sid = lax.axis_index("s")
        wid = cid * NS + sid
        base = wid * per_w

        def fire_ie(jj, b):
            pltpu.async_copy(eidx_hbm.at[wid, jj, 0], sv[b], si[b])
            pltpu.async_copy(eidx_hbm.at[wid, jj, 1], dv[b], sd[b])
            pltpu.async_copy(e_hbm.at[pl.ds(base + jj * chunk, chunk), :],
                             ev[b], se[b])

        def fire_gather(jj, b):
            pltpu.make_async_copy(eidx_hbm.at[wid, jj, 0], sv[b], si[b]).wait()
            pltpu.async_copy(h_hbm.at[sv[b]], xg[b], sg[b])

        for c in range(nbuf - 1):
            fire_ie(c, c)
        for c in range(2):
            fire_gather(c, c)

        r0 = sid * rows_per_sub
        pltpu.sync_copy(z_hbm.at[pl.ds(r0, rows_per_sub), :],
                        agg_sh.at[pl.ds(r0, rows_per_sub), :])
        plsc.subcore_barrier()

        @pl.loop(0, n_chunks, step=nbuf)
        def _(j0):
            for b in range(nbuf):
                jj = j0 + b
                bm1 = (b + nbuf - 1) % nbuf
                b2 = (b + 2) % nbuf

                @pl.when(jj >= 1)
                def _():
                    pltpu.make_async_copy(
                        xg[bm1], agg_sh.at[dv[bm1]], ss[bm1]).wait()

                @pl.when(jj + nbuf - 1 < n_chunks)
                def _():
                    fire_ie(jj + nbuf - 1, bm1)

                @pl.when(jj + 2 < n_chunks)
                def _():
                    fire_gather(jj + 2, b2)

                pltpu.make_async_copy(
                    e_hbm.at[pl.ds(base, chunk), :], ev[b], se[b]).wait()
                pltpu.make_async_copy(
                    eidx_hbm.at[wid, jj, 1], dv[b], sd[b]).wait()
                pltpu.make_async_copy(
                    h_hbm.at[sv[b]], xg[b], sg[b]).wait()

                @plsc.parallel_loop(0, chunk)
                def _(r):
                    for c0 in range(0, D, LANES):
                        v = xg[b][r, pl.ds(c0, LANES)] \
                            + ev[b][r, pl.ds(c0, LANES)]
                        xg[b][r, pl.ds(c0, LANES)] = jnp.maximum(v, 0.0)

                pltpu.async_copy(xg[b], agg_sh.at[dv[b]], ss[b],
                                 add=True)

        lb = (n_chunks - 1) % nbuf
        pltpu.make_async_copy(
            xg[lb], agg_sh.at[dv[lb]], ss[lb]).wait()

        plsc.subcore_barrier()
        pltpu.sync_copy(agg_sh.at[pl.ds(r0, rows_per_sub), :],
                        out_hbm.at[cid, pl.ds(r0, rows_per_sub), :])

    return k(h, e, eidx, zeros_nd)


def kernel(x, edge_index, edge_attr, params):
    chunk, nbuf = 40, 4
    Nn, D = x.shape
    E = edge_index.shape[1]
    NW = NC * NS
    Np = ((Nn + 8 * NS - 1) // (8 * NS)) * (8 * NS)
    step = chunk * nbuf
    while (NW * step) % 2048:
        step *= 2
    per_w = -(-E // (NW * step)) * step
    Ep = NW * per_w
    if Ep > E:
        pad_ar = jnp.arange(Ep - E, dtype=jnp.int32)
        edge_index = jnp.concatenate(
            [edge_index,
             jnp.stack([pad_ar % Nn, Nn + pad_ar % (Np - Nn)])], axis=1)
        edge_attr = jnp.concatenate(
            [edge_attr, jnp.zeros((Ep - E, edge_attr.shape[1]),
                                  edge_attr.dtype)])
    n_chunks = per_w // chunk
    eidx = edge_index.reshape(2, NW, n_chunks, chunk).transpose(1, 2, 0, 3)
    zeros_nd = jnp.zeros((Np, D), jnp.float32)

    es = [_edge_linear(edge_attr, We, be) for (We, be, *_rest) in params]

    h = x
    xs = [x]
    for l, (We, be, W1, b1, W2, b2, alpha) in enumerate(params):
        agg2 = _sc_message_pass(h, es[l], eidx, zeros_nd,
                                chunk=chunk, nbuf=nbuf)
        h = _node_mlp(h, agg2[0], agg2[1], W1, b1, W2, b2, alpha)
        xs.append(h)
    return jnp.concatenate(xs, axis=-1)


# speedup vs baseline: 2.1814x; 1.0112x over previous
import functools

import jax
import jax.numpy as jnp
from jax import lax
from jax.experimental import pallas as pl
from jax.experimental.pallas import tpu as pltpu
from jax.experimental.pallas import tpu_sc as plsc

NC = 2
NS = 16
LANES = 16


def _edge_linear_body(ea_ref, we_ref, be_ref, out_ref):
    out_ref[...] = (
        jnp.dot(ea_ref[...], we_ref[...], preferred_element_type=jnp.float32)
        + be_ref[...]
    )


def _edge_linear(edge_attr, We, be, block=8192):
    E, DE = edge_attr.shape
    D = We.shape[1]
    while E % block:
        block //= 2
    assert E % block == 0
    return pl.pallas_call(
        _edge_linear_body,
        grid=(E // block,),
        in_specs=[
            pl.BlockSpec((block, DE), lambda i: (i, 0)),
            pl.BlockSpec((DE, D), lambda i: (0, 0)),
            pl.BlockSpec((1, D), lambda i: (0, 0)),
        ],
        out_specs=pl.BlockSpec((block, D), lambda i: (i, 0)),
        out_shape=jax.ShapeDtypeStruct((E, D), jnp.float32),
    )(edge_attr, We, be.reshape(1, D))


def _node_mlp_body(h_ref, a0_ref, a1_ref, w1_ref, b1_ref, w2_ref, b2_ref,
                   alpha_ref, out_ref):
    h = h_ref[...]
    s = h + a0_ref[...] + a1_ref[...]
    t = jnp.dot(s, w1_ref[...], preferred_element_type=jnp.float32) + b1_ref[...]
    t = jnp.maximum(t, 0.0)
    xn = jnp.dot(t, w2_ref[...], preferred_element_type=jnp.float32) + b2_ref[...]
    a = alpha_ref[0, 0]
    out_ref[...] = a * h + (1.0 - a) * xn


def _node_mlp(h, a0, a1, W1, b1, W2, b2, alpha, block=2000):
    Nn, D = h.shape
    assert Nn % block == 0
    return pl.pallas_call(
        _node_mlp_body,
        grid=(Nn // block,),
        in_specs=[
            pl.BlockSpec((block, D), lambda i: (i, 0)),
            pl.BlockSpec((block, D), lambda i: (i, 0)),
            pl.BlockSpec((block, D), lambda i: (i, 0)),
            pl.BlockSpec((D, D), lambda i: (0, 0)),
            pl.BlockSpec((1, D), lambda i: (0, 0)),
            pl.BlockSpec((D, D), lambda i: (0, 0)),
            pl.BlockSpec((1, D), lambda i: (0, 0)),
            pl.BlockSpec((1, 1), lambda i: (0, 0)),
        ],
        out_specs=pl.BlockSpec((block, D), lambda i: (i, 0)),
        out_shape=jax.ShapeDtypeStruct((Nn, D), jnp.float32),
    )(h, a0, a1, W1, b1.reshape(1, D), W2, b2.reshape(1, D),
      alpha.reshape(1, 1))


def _sc_message_pass(h, e, eidx, zeros_nd, chunk=16, nbuf=5):
    E, D = e.shape
    Np = zeros_nd.shape[0]
    NW = NC * NS
    per_w = E // NW
    assert per_w * NW == E and per_w % chunk == 0 and chunk % 8 == 0
    n_chunks = per_w // chunk
    assert n_chunks % nbuf == 0 and nbuf >= 3
    assert eidx.shape == (NW, n_chunks, 2, chunk)
    rows_per_sub = Np // NS
    assert rows_per_sub * NS == Np and rows_per_sub % 8 == 0

    mesh = plsc.VectorSubcoreMesh(core_axis_name="c", subcore_axis_name="s")

    @functools.partial(
        pl.kernel,
        out_type=jax.ShapeDtypeStruct((NC, Np, D), jnp.float32),
        mesh=mesh,
        scratch_types=(
            [pltpu.VMEM((chunk,), jnp.int32) for _ in range(2 * nbuf)]
            + [pltpu.VMEM((chunk, D), jnp.float32) for _ in range(2 * nbuf)]
            + [pltpu.VMEM_SHARED((Np, D), jnp.float32)]
            + [pltpu.SemaphoreType.DMA for _ in range(5 * nbuf)]
        ),
    )
    def k(h_hbm, e_hbm, eidx_hbm, z_hbm, out_hbm, *rest):
        sv = rest[0:nbuf]
        dv = rest[nbuf:2 * nbuf]
        xg = rest[2 * nbuf:3 * nbuf]
        ev = rest[3 * nbuf:4 * nbuf]
        agg_sh = rest[4 * nbuf]
        si = rest[4 * nbuf + 1:4 * nbuf + 1 + nbuf]
        sd = rest[4 * nbuf + 1 + nbuf:4 * nbuf + 1 + 2 * nbuf]
        se = rest[4 * nbuf + 1 + 2 * nbuf:4 * nbuf + 1 + 3 * nbuf]
        sg = rest[4 * nbuf + 1 + 3 * nbuf:4 * nbuf + 1 + 4 * nbuf]
        ss = rest[4 * nbuf + 1 + 4 * nbuf:4 * nbuf + 1 + 5 * nbuf]

        cid = lax.axis_index("c")
        sid = lax.axis_index("s")
        wid = cid * NS + sid
        base = wid * per_w

        def fire_ie(jj, b):
            pltpu.async_copy(eidx_hbm.at[wid, jj, 0], sv[b], si[b])
            pltpu.async_copy(eidx_hbm.at[wid, jj, 1], dv[b], sd[b])
            pltpu.async_copy(e_hbm.at[pl.ds(base + jj * chunk, chunk), :],
                             ev[b], se[b])

        def fire_gather(jj, b):
            pltpu.make_async_copy(eidx_hbm.at[wid, jj, 0], sv[b], si[b]).wait()
            pltpu.async_copy(h_hbm.at[sv[b]], xg[b], sg[b])

        for c in range(nbuf - 1):
            fire_ie(c, c)
        for c in range(2):
            fire_gather(c, c)

        r0 = sid * rows_per_sub
        pltpu.sync_copy(z_hbm.at[pl.ds(r0, rows_per_sub), :],
                        agg_sh.at[pl.ds(r0, rows_per_sub), :])
        plsc.subcore_barrier()

        @pl.loop(0, n_chunks, step=nbuf)
        def _(j0):
            for b in range(nbuf):
                jj = j0 + b
                bm1 = (b + nbuf - 1) % nbuf
                b2 = (b + 2) % nbuf

                @pl.when(jj >= 1)
                def _():
                    pltpu.make_async_copy(
                        xg[bm1], agg_sh.at[dv[bm1]], ss[bm1]).wait()

                @pl.when(jj + nbuf - 1 < n_chunks)
                def _():
                    fire_ie(jj + nbuf - 1, bm1)

                @pl.when(jj + 2 < n_chunks)
                def _():
                    fire_gather(jj + 2, b2)

                pltpu.make_async_copy(
                    e_hbm.at[pl.ds(base, chunk), :], ev[b], se[b]).wait()
                pltpu.make_async_copy(
                    eidx_hbm.at[wid, jj, 1], dv[b], sd[b]).wait()
                pltpu.make_async_copy(
                    h_hbm.at[sv[b]], xg[b], sg[b]).wait()

                @plsc.parallel_loop(0, chunk)
                def _(r):
                    for c0 in range(0, D, LANES):
                        v = xg[b][r, pl.ds(c0, LANES)] \
                            + ev[b][r, pl.ds(c0, LANES)]
                        xg[b][r, pl.ds(c0, LANES)] = jnp.maximum(v, 0.0)

                pltpu.async_copy(xg[b], agg_sh.at[dv[b]], ss[b],
                                 add=True)

        lb = (n_chunks - 1) % nbuf
        pltpu.make_async_copy(
            xg[lb], agg_sh.at[dv[lb]], ss[lb]).wait()

        plsc.subcore_barrier()
        pltpu.sync_copy(agg_sh.at[pl.ds(r0, rows_per_sub), :],
                        out_hbm.at[cid, pl.ds(r0, rows_per_sub), :])

    return k(h, e, eidx, zeros_nd)


def kernel(x, edge_index, edge_attr, params):
    chunk, nbuf = 40, 4
    Nn, D = x.shape
    E = edge_index.shape[1]
    NW = NC * NS
    Np = ((Nn + 8 * NS - 1) // (8 * NS)) * (8 * NS)
    step = chunk * nbuf
    while (NW * step) % 2048:
        step *= 2
    per_w = -(-E // (NW * step)) * step
    Ep = NW * per_w
    if Ep > E:
        pad_ar = jnp.arange(Ep - E, dtype=jnp.int32)
        edge_index = jnp.concatenate(
            [edge_index,
             jnp.stack([pad_ar % Nn, Nn + pad_ar % (Np - Nn)])], axis=1)
        edge_attr = jnp.concatenate(
            [edge_attr, jnp.zeros((Ep - E, edge_attr.shape[1]),
                                  edge_attr.dtype)])
    n_chunks = per_w // chunk
    eidx = edge_index.reshape(2, NW, n_chunks, chunk).transpose(1, 2, 0, 3)
    zeros_nd = jnp.zeros((Np, D), jnp.float32)

    es = [_edge_linear(edge_attr, We, be) for (We, be, *_rest) in params]

    h = x
    xs = [x]
    for l, (We, be, W1, b1, W2, b2, alpha) in enumerate(params):
        agg2 = _sc_message_pass(h, es[l], eidx, zeros_nd,
                                chunk=chunk, nbuf=nbuf)
        h = _node_mlp(h, agg2[0], agg2[1], W1, b1, W2, b2, alpha)
        xs.append(h)
    return jnp.concatenate(xs, axis=-1)
